# baseline (speedup 1.0000x reference)
import functools

import numpy as np

import jax
import jax.numpy as jnp
from jax import lax
from jax.experimental import pallas as pl
from jax.experimental.pallas import tpu as pltpu
from jax.experimental.pallas import tpu_sc as plsc

_NC = 2
_NS = 16
_NW = _NC * _NS


def _gather0_call(n_pad, e_pad, chunk):
    epw = e_pad // _NW
    n_chunks = epw // chunk
    sub = chunk // 128
    mesh = plsc.VectorSubcoreMesh(core_axis_name="c", subcore_axis_name="s")

    @functools.partial(
        pl.kernel,
        out_type=(jax.ShapeDtypeStruct((e_pad, 8), jnp.float32),
                  jax.ShapeDtypeStruct((e_pad, 8), jnp.float32)),
        mesh=mesh,
        compiler_params=pltpu.CompilerParams(use_tc_tiling_on_sc=False),
        scratch_types=[
            pltpu.VMEM((sub, 128), jnp.int32),
            pltpu.VMEM((sub, 128), jnp.int32),
            pltpu.VMEM((chunk, 8), jnp.float32),
            pltpu.VMEM((chunk, 8), jnp.float32),
            pltpu.SemaphoreType.DMA,
        ],
    )
    def k(t_hbm, ip_hbm, jp_hbm, oi_hbm, oj_hbm, ipv, jpv, ri, rj, sem):
        wid = lax.axis_index("s") * _NC + lax.axis_index("c")
        wb128 = wid * (epw // 128)

        def body(kk, carry):
            b128 = wb128 + kk * sub
            base = b128 * 128
            pltpu.sync_copy(ip_hbm.at[pl.ds(b128, sub)], ipv)
            pltpu.sync_copy(jp_hbm.at[pl.ds(b128, sub)], jpv)
            cps = []
            for u in range(sub):
                cps.append(pltpu.async_copy(
                    t_hbm.at[ipv.at[u]], ri.at[pl.ds(u * 128, 128)], sem))
                cps.append(pltpu.async_copy(
                    t_hbm.at[jpv.at[u]], rj.at[pl.ds(u * 128, 128)], sem))
            for cp in cps:
                cp.wait()
            pltpu.sync_copy(ri, oi_hbm.at[pl.ds(base, chunk)])
            pltpu.sync_copy(rj, oj_hbm.at[pl.ds(base, chunk)])
            return carry

        lax.fori_loop(0, n_chunks, body, 0)

    return k


def _gather1_call(n_pad, e_pad, chunk):
    epw = e_pad // _NW
    n_chunks = epw // chunk
    sub = chunk // 128
    mesh = plsc.VectorSubcoreMesh(core_axis_name="c", subcore_axis_name="s")

    @functools.partial(
        pl.kernel,
        out_type=(jax.ShapeDtypeStruct((e_pad, 8), jnp.float32),
                  jax.ShapeDtypeStruct((e_pad, 32), jnp.float32),
                  jax.ShapeDtypeStruct((e_pad, 8), jnp.float32),
                  jax.ShapeDtypeStruct((e_pad, 32), jnp.float32)),
        mesh=mesh,
        compiler_params=pltpu.CompilerParams(use_tc_tiling_on_sc=False),
        scratch_types=[
            pltpu.VMEM((sub, 128), jnp.int32),
            pltpu.VMEM((sub, 128), jnp.int32),
            pltpu.VMEM((chunk, 8), jnp.float32),
            pltpu.VMEM((chunk, 32), jnp.float32),
            pltpu.VMEM((chunk, 8), jnp.float32),
            pltpu.VMEM((chunk, 32), jnp.float32),
            pltpu.SemaphoreType.DMA,
        ],
    )
    def k(tc_hbm, ta_hbm, tb_hbm, ip_hbm, jp_hbm,
          ci_hbm, xa_hbm, cj_hbm, xb_hbm,
          ipv, jpv, rci, rxa, rcj, rxb, sem):
        wid = lax.axis_index("s") * _NC + lax.axis_index("c")
        wb128 = wid * (epw // 128)

        def body(kk, carry):
            b128 = wb128 + kk * sub
            base = b128 * 128
            pltpu.sync_copy(ip_hbm.at[pl.ds(b128, sub)], ipv)
            pltpu.sync_copy(jp_hbm.at[pl.ds(b128, sub)], jpv)
            cps = []
            for u in range(sub):
                sl = pl.ds(u * 128, 128)
                cps.append(pltpu.async_copy(tc_hbm.at[ipv.at[u]], rci.at[sl], sem))
                cps.append(pltpu.async_copy(ta_hbm.at[ipv.at[u]], rxa.at[sl], sem))
                cps.append(pltpu.async_copy(tc_hbm.at[jpv.at[u]], rcj.at[sl], sem))
                cps.append(pltpu.async_copy(tb_hbm.at[jpv.at[u]], rxb.at[sl], sem))
            for cp in cps:
                cp.wait()
            pltpu.sync_copy(rci, ci_hbm.at[pl.ds(base, chunk)])
            pltpu.sync_copy(rxa, xa_hbm.at[pl.ds(base, chunk)])
            pltpu.sync_copy(rcj, cj_hbm.at[pl.ds(base, chunk)])
            pltpu.sync_copy(rxb, xb_hbm.at[pl.ds(base, chunk)])
            return carry

        lax.fori_loop(0, n_chunks, body, 0)

    return k


def _scatter_call(n, e_pad, d, chunk, feature_split):
    n_pad = n + 32
    zr = n_pad // _NS
    orow = n // _NS
    ept = e_pad // _NS if feature_split else e_pad // _NW
    n_chunks = ept // chunk
    sub = chunk // 128
    mesh = plsc.VectorSubcoreMesh(core_axis_name="c", subcore_axis_name="s")

    @functools.partial(
        pl.kernel,
        out_type=jax.ShapeDtypeStruct((_NC, n, d), jnp.float32),
        mesh=mesh,
        compiler_params=pltpu.CompilerParams(use_tc_tiling_on_sc=False),
        scratch_types=[
            pltpu.VMEM((sub, 128), jnp.int32),
            pltpu.VMEM((chunk, d), jnp.float32),
            pltpu.VMEM_SHARED((n_pad, d), jnp.float32),
            pltpu.SemaphoreType.DMA,
        ],
    )
    def k(data_hbm, ip_hbm, z_hbm, out_hbm, iv, dv, acc, sem):
        c = lax.axis_index("c")
        s = lax.axis_index("s")
        pltpu.sync_copy(z_hbm.at[pl.ds(s * zr, zr)], acc.at[pl.ds(s * zr, zr)])
        plsc.subcore_barrier()
        if feature_split:
            tb128 = s * (ept // 128)
        else:
            tb128 = (s * _NC + c) * (ept // 128)

        def body(kk, carry):
            b128 = tb128 + kk * sub
            base = b128 * 128
            pltpu.sync_copy(ip_hbm.at[pl.ds(b128, sub)], iv)
            if feature_split:
                pltpu.sync_copy(
                    data_hbm.at[pl.ds(base, chunk), pl.ds(c * d, d)], dv)
            else:
                pltpu.sync_copy(data_hbm.at[pl.ds(base, chunk)], dv)
            adds = []
            for u in range(sub):
                adds.append(pltpu.async_copy(
                    dv.at[pl.ds(u * 128, 128)], acc.at[iv.at[u]], sem, add=True))
            for a in adds:
                a.wait()
            return carry

        lax.fori_loop(0, n_chunks, body, 0)
        plsc.subcore_barrier()
        pltpu.sync_copy(acc.at[pl.ds(s * orow, orow)],
                        out_hbm.at[c].at[pl.ds(s * orow, orow)])

    return k


def _silu(v):
    return v * jax.nn.sigmoid(v)


def _ea_unfold(ea_blk, ps, qs):
    acc = None
    for s in range(8):
        t = jnp.dot(jnp.dot(ps[s], ea_blk, preferred_element_type=jnp.float32),
                    qs[s], preferred_element_type=jnp.float32)
        acc = t if acc is None else acc + t
    return acc


def _edge0_body(g0i, g0j, ea, ps, qs, wsel, wia, wjb, wdd, b1t, we2bd, b2t,
                wc1bd, bc1t, wc2s, bc2, sel, onep, out_m, out_v):
    gi = g0i[...]
    gj = g0j[...]
    diff = gi - gj
    d2 = diff * diff
    eau = _ea_unfold(ea[...], ps[...], qs[...])
    hpre = (jnp.dot(gi, wia[...], preferred_element_type=jnp.float32)
            + jnp.dot(gj, wjb[...], preferred_element_type=jnp.float32)
            + jnp.dot(d2, wdd[...], preferred_element_type=jnp.float32)
            + jnp.dot(eau, wsel[...], preferred_element_type=jnp.float32)
            + b1t[...])
    m = _silu(hpre)
    m = _silu(jnp.dot(m.astype(jnp.bfloat16), we2bd[...],
                      preferred_element_type=jnp.float32) + b2t[...])
    q = _silu(jnp.dot(m.astype(jnp.bfloat16), wc1bd[...],
                      preferred_element_type=jnp.float32) + bc1t[...])
    cwp = (jnp.dot(q, wc2s[...], preferred_element_type=jnp.float32)
           + bc2[...])
    rcw = diff * jnp.dot(cwp, sel[...], preferred_element_type=jnp.float32)
    out_m[...] = m
    out_v[...] = rcw + onep[...]


def _edge1_body(ci, xai, cj, xbj, ea, ps, qs, wsel, wdd, b1t, we2bd, b2t,
                wc1bd, bc1t, wc2s, bc2, sel, out):
    diff = ci[...] - cj[...]
    d2 = diff * diff
    eau = _ea_unfold(ea[...], ps[...], qs[...])
    hpre = (xai[...] + xbj[...]
            + jnp.dot(eau, wsel[...], preferred_element_type=jnp.float32)
            + jnp.dot(d2, wdd[...], preferred_element_type=jnp.float32)
            + b1t[...])
    m = _silu(hpre)
    m = _silu(jnp.dot(m.astype(jnp.bfloat16), we2bd[...],
                      preferred_element_type=jnp.float32) + b2t[...])
    q = _silu(jnp.dot(m.astype(jnp.bfloat16), wc1bd[...],
                      preferred_element_type=jnp.float32) + bc1t[...])
    cwp = (jnp.dot(q, wc2s[...], preferred_element_type=jnp.float32)
           + bc2[...])
    out[...] = diff * jnp.dot(cwp, sel[...],
                              preferred_element_type=jnp.float32)


def _wspec(shape):
    return pl.BlockSpec(shape, lambda i: tuple(0 for _ in shape))


def _edge0_call(e_pad, be, weights):
    grid = (e_pad // be,)
    bp = be // 16
    wspecs = [_wspec(w.shape) for w in weights]
    return pl.pallas_call(
        _edge0_body,
        grid=grid,
        in_specs=[pl.BlockSpec((bp, 128), lambda i: (i, 0)),
                  pl.BlockSpec((bp, 128), lambda i: (i, 0)),
                  pl.BlockSpec((bp // 8, 128), lambda i: (i, 0))] + wspecs,
        out_specs=[pl.BlockSpec((bp, 512), lambda i: (i, 0)),
                   pl.BlockSpec((bp, 128), lambda i: (i, 0))],
        out_shape=[jax.ShapeDtypeStruct((e_pad // 16, 512), jnp.float32),
                   jax.ShapeDtypeStruct((e_pad // 16, 128), jnp.float32)],
    )


def _edge1_call(e_pad, be, weights):
    grid = (e_pad // be,)
    bp = be // 16
    wspecs = [_wspec(w.shape) for w in weights]
    return pl.pallas_call(
        _edge1_body,
        grid=grid,
        in_specs=[pl.BlockSpec((bp, 128), lambda i: (i, 0)),
                  pl.BlockSpec((bp, 512), lambda i: (i, 0)),
                  pl.BlockSpec((bp, 128), lambda i: (i, 0)),
                  pl.BlockSpec((bp, 512), lambda i: (i, 0)),
                  pl.BlockSpec((bp // 8, 128), lambda i: (i, 0))] + wspecs,
        out_specs=pl.BlockSpec((bp, 128), lambda i: (i, 0)),
        out_shape=jax.ShapeDtypeStruct((e_pad // 16, 128), jnp.float32),
    )


def _node_body(x, pos, accm, accv, wn1f, wn1m_lo, wn1m_hi, bn1, wn2, bn2,
               w1a1, w1b1, c8, xa1, xb1, cnt):
    v = accv[0] + accv[1]
    c = jnp.maximum(v[:, 2:3], 1.0)
    cnt[...] = c
    coors1 = pos[...] + v[:, 0:2] / c
    c8[...] = jnp.concatenate(
        [coors1, jnp.zeros((coors1.shape[0], 6), jnp.float32)], axis=1)
    h = _silu(jnp.dot(x[...], wn1f[...], preferred_element_type=jnp.float32)
              + jnp.dot(accm[0], wn1m_lo[...],
                        preferred_element_type=jnp.float32)
              + jnp.dot(accm[1], wn1m_hi[...],
                        preferred_element_type=jnp.float32)
              + bn1[...])
    feats1 = jnp.dot(h, wn2[...], preferred_element_type=jnp.float32) + bn2[...]
    xa1[...] = jnp.dot(feats1, w1a1[...], preferred_element_type=jnp.float32)
    xb1[...] = jnp.dot(feats1, w1b1[...], preferred_element_type=jnp.float32)


def _node_call(n, bn, weights):
    grid = (n // bn,)
    wspecs = [_wspec(w.shape) for w in weights]
    return pl.pallas_call(
        _node_body,
        grid=grid,
        in_specs=[pl.BlockSpec((bn, 2), lambda i: (i, 0)),
                  pl.BlockSpec((bn, 2), lambda i: (i, 0)),
                  pl.BlockSpec((2, bn, 16), lambda i: (0, i, 0)),
                  pl.BlockSpec((2, bn, 8), lambda i: (0, i, 0))] + wspecs,
        out_specs=[pl.BlockSpec((bn, 8), lambda i: (i, 0)),
                   pl.BlockSpec((bn, 32), lambda i: (i, 0)),
                   pl.BlockSpec((bn, 32), lambda i: (i, 0)),
                   pl.BlockSpec((bn, 1), lambda i: (i, 0))],
        out_shape=[jax.ShapeDtypeStruct((n, 8), jnp.float32),
                   jax.ShapeDtypeStruct((n, 32), jnp.float32),
                   jax.ShapeDtypeStruct((n, 32), jnp.float32),
                   jax.ShapeDtypeStruct((n, 1), jnp.float32)],
    )


def _final_body(c8, acc, cnt, out):
    v = acc[0] + acc[1]
    out[...] = c8[:, 0:2] + v[:, 0:2] / cnt[...]


def _final_call(n, bn):
    grid = (n // bn,)
    return pl.pallas_call(
        _final_body,
        grid=grid,
        in_specs=[pl.BlockSpec((bn, 8), lambda i: (i, 0)),
                  pl.BlockSpec((2, bn, 8), lambda i: (0, i, 0)),
                  pl.BlockSpec((bn, 1), lambda i: (i, 0))],
        out_specs=pl.BlockSpec((bn, 2), lambda i: (i, 0)),
        out_shape=jax.ShapeDtypeStruct((n, 2), jnp.float32),
    )


def kernel(x, edge_index, edge_attr, batch, positions, params):
    n = x.shape[0]
    e = edge_index.shape[1]
    assert n % 32 == 0 and n % 16 == 0
    quantum = 32 * 2048
    e_pad = ((e + quantum - 1) // quantum) * quantum
    n_pad = n + 32

    f32 = jnp.float32
    i = edge_index[0]
    j = edge_index[1]
    pad_e = e_pad - e
    ip = jnp.concatenate([i, jnp.full((pad_e,), n, jnp.int32)])
    jp = jnp.concatenate([j, jnp.full((pad_e,), n, jnp.int32)])
    ip2 = ip.reshape(e_pad // 128, 128)
    jp2 = jp.reshape(e_pad // 128, 128)
    ea16 = jnp.pad(edge_attr.astype(f32).reshape(-1),
                   (0, pad_e)).reshape(e_pad // 128, 128)

    p0 = params["l0"]
    p1 = params["l1"]

    w1a0 = p0["W_e1"][0:2]
    w1b0 = p0["W_e1"][2:4]
    wd0 = p0["W_e1"][4:5]
    wa0 = p0["W_e1"][5:6]
    b10 = p0["b_e1"].reshape(1, 32)
    we20, b20 = p0["W_e2"], p0["b_e2"].reshape(1, 32)
    wc10, bc10 = p0["W_c1"], p0["b_c1"].reshape(1, 32)
    wc20, bc20 = p0["W_c2"], p0["b_c2"].reshape(1, 1)
    wn1f = p0["W_n1"][0:2]
    wn1m_lo = p0["W_n1"][2:18]
    wn1m_hi = p0["W_n1"][18:34]
    bn1 = p0["b_n1"].reshape(1, 32)
    wn2, bn2 = p0["W_n2"], p0["b_n2"].reshape(1, 32)

    w1a1 = p1["W_e1"][0:32]
    w1b1 = p1["W_e1"][32:64]
    wd1 = p1["W_e1"][64:65]
    wa1 = p1["W_e1"][65:66]
    b11 = p1["b_e1"].reshape(1, 32)
    we21, b21 = p1["W_e2"], p1["b_e2"].reshape(1, 32)
    wc11, bc11 = p1["W_c1"], p1["b_c1"].reshape(1, 32)
    wc21, bc21 = p1["W_c2"], p1["b_c2"].reshape(1, 1)

    eye16 = jnp.eye(16, dtype=f32)

    def bd(blk):
        return jnp.kron(eye16, blk)

    def t16(row):
        return jnp.tile(row, (1, 16))

    z8x32 = jnp.zeros((8, 32), f32)
    wia0 = bd(z8x32.at[2:4].set(w1a0))
    wjb0 = bd(z8x32.at[2:4].set(w1b0))
    wdd0 = bd(z8x32.at[0].set(wd0[0]).at[1].set(wd0[0]))
    wdd1 = bd(z8x32.at[0].set(wd1[0]).at[1].set(wd1[0]))
    sel = bd(jnp.array([[1, 1, 0, 0, 0, 0, 0, 0]], f32))
    onep = t16(jnp.array([[0, 0, 1, 0, 0, 0, 0, 0]], f32))
    psn = np.zeros((8, 128, 16), np.float32)
    qsn = np.zeros((8, 128, 16), np.float32)
    for s_ in range(8):
        for q_ in range(16):
            psn[s_, 8 * q_ + s_, q_] = 1.0
        for g_ in range(16):
            qsn[s_, 16 * s_ + g_, g_] = 1.0
    ps = jnp.asarray(psn)
    qs = jnp.asarray(qsn)
    wsel0 = bd(wa0)
    wsel1 = bd(wa1)

    t0 = jnp.pad(jnp.concatenate([positions.astype(f32), x.astype(f32)],
                                 axis=1), ((0, 32), (0, 4)))
    g0i, g0j = _gather0_call(n_pad, e_pad, 2048)(t0, ip2, jp2)

    w_edge0 = (ps, qs, wsel0, wia0, wjb0, wdd0, t16(b10),
               bd(we20).astype(jnp.bfloat16), t16(b20),
               bd(wc10).astype(jnp.bfloat16), t16(bc10), bd(wc20), bc20,
               sel, onep)
    m2p, v0p = _edge0_call(e_pad, 2048, w_edge0)(
        g0i.reshape(e_pad // 16, 128), g0j.reshape(e_pad // 16, 128),
        ea16, *w_edge0)
    m2 = m2p.reshape(e_pad, 32)
    v0 = v0p.reshape(e_pad, 8)

    z16 = jnp.zeros((n_pad, 16), f32)
    z8 = jnp.zeros((n_pad, 8), f32)
    accm = _scatter_call(n, e_pad, 16, 1024, True)(m2, ip2, z16)
    accv = _scatter_call(n, e_pad, 8, 2048, False)(v0, ip2, z8)

    w_node = (wn1f, wn1m_lo, wn1m_hi, bn1, wn2, bn2, w1a1, w1b1)
    c8, xa1, xb1, cnt = _node_call(n, 2000, w_node)(
        x.astype(f32), positions.astype(f32), accm, accv, *w_node)

    tc1 = jnp.pad(c8, ((0, 32), (0, 0)))
    ta1 = jnp.pad(xa1, ((0, 32), (0, 0)))
    tb1 = jnp.pad(xb1, ((0, 32), (0, 0)))
    ci, xai, cj, xbj = _gather1_call(n_pad, e_pad, 1024)(
        tc1, ta1, tb1, ip2, jp2)

    w_edge1 = (ps, qs, wsel1, wdd1, t16(b11),
               bd(we21).astype(jnp.bfloat16), t16(b21),
               bd(wc11).astype(jnp.bfloat16), t16(bc11), bd(wc21), bc21,
               sel)
    v1p = _edge1_call(e_pad, 2048, w_edge1)(
        ci.reshape(e_pad // 16, 128), xai.reshape(e_pad // 16, 512),
        cj.reshape(e_pad // 16, 128), xbj.reshape(e_pad // 16, 512),
        ea16, *w_edge1)
    v1 = v1p.reshape(e_pad, 8)

    acc1 = _scatter_call(n, e_pad, 8, 2048, False)(v1, ip2, z8)

    return _final_call(n, 2000)(c8, acc1, cnt)


# speedup vs baseline: 1.0103x; 1.0103x over previous
import functools

import numpy as np

import jax
import jax.numpy as jnp
from jax import lax
from jax.experimental import pallas as pl
from jax.experimental.pallas import tpu as pltpu
from jax.experimental.pallas import tpu_sc as plsc

_NC = 2
_NS = 16
_NW = _NC * _NS


def _gather0_call(n_pad, e_pad, chunk):
    epw = e_pad // _NW
    n_chunks = epw // chunk
    sub = chunk // 128
    mesh = plsc.VectorSubcoreMesh(core_axis_name="c", subcore_axis_name="s")

    @functools.partial(
        pl.kernel,
        out_type=(jax.ShapeDtypeStruct((e_pad, 8), jnp.float32),
                  jax.ShapeDtypeStruct((e_pad, 8), jnp.float32)),
        mesh=mesh,
        compiler_params=pltpu.CompilerParams(use_tc_tiling_on_sc=False),
        scratch_types=[
            pltpu.VMEM((sub, 128), jnp.int32),
            pltpu.VMEM((sub, 128), jnp.int32),
            pltpu.VMEM((chunk, 8), jnp.float32),
            pltpu.VMEM((chunk, 8), jnp.float32),
            pltpu.SemaphoreType.DMA,
        ],
    )
    def k(t_hbm, ip_hbm, jp_hbm, oi_hbm, oj_hbm, ipv, jpv, ri, rj, sem):
        wid = lax.axis_index("s") * _NC + lax.axis_index("c")
        wb128 = wid * (epw // 128)

        def body(kk, carry):
            b128 = wb128 + kk * sub
            base = b128 * 128
            pltpu.sync_copy(ip_hbm.at[pl.ds(b128, sub)], ipv)
            pltpu.sync_copy(jp_hbm.at[pl.ds(b128, sub)], jpv)
            cps = []
            for u in range(sub):
                cps.append(pltpu.async_copy(
                    t_hbm.at[ipv.at[u]], ri.at[pl.ds(u * 128, 128)], sem))
                cps.append(pltpu.async_copy(
                    t_hbm.at[jpv.at[u]], rj.at[pl.ds(u * 128, 128)], sem))
            for cp in cps:
                cp.wait()
            pltpu.sync_copy(ri, oi_hbm.at[pl.ds(base, chunk)])
            pltpu.sync_copy(rj, oj_hbm.at[pl.ds(base, chunk)])
            return carry

        lax.fori_loop(0, n_chunks, body, 0)

    return k


def _gather1_call(n_pad, e_pad, chunk):
    epw = e_pad // _NW
    n_chunks = epw // chunk
    sub = chunk // 128
    mesh = plsc.VectorSubcoreMesh(core_axis_name="c", subcore_axis_name="s")

    @functools.partial(
        pl.kernel,
        out_type=(jax.ShapeDtypeStruct((e_pad, 8), jnp.float32),
                  jax.ShapeDtypeStruct((e_pad, 32), jnp.float32),
                  jax.ShapeDtypeStruct((e_pad, 8), jnp.float32),
                  jax.ShapeDtypeStruct((e_pad, 32), jnp.float32)),
        mesh=mesh,
        compiler_params=pltpu.CompilerParams(use_tc_tiling_on_sc=False),
        scratch_types=[
            pltpu.VMEM((sub, 128), jnp.int32),
            pltpu.VMEM((sub, 128), jnp.int32),
            pltpu.VMEM((chunk, 8), jnp.float32),
            pltpu.VMEM((chunk, 32), jnp.float32),
            pltpu.VMEM((chunk, 8), jnp.float32),
            pltpu.VMEM((chunk, 32), jnp.float32),
            pltpu.SemaphoreType.DMA,
        ],
    )
    def k(tc_hbm, ta_hbm, tb_hbm, ip_hbm, jp_hbm,
          ci_hbm, xa_hbm, cj_hbm, xb_hbm,
          ipv, jpv, rci, rxa, rcj, rxb, sem):
        wid = lax.axis_index("s") * _NC + lax.axis_index("c")
        wb128 = wid * (epw // 128)

        def body(kk, carry):
            b128 = wb128 + kk * sub
            base = b128 * 128
            pltpu.sync_copy(ip_hbm.at[pl.ds(b128, sub)], ipv)
            pltpu.sync_copy(jp_hbm.at[pl.ds(b128, sub)], jpv)
            cps = []
            for u in range(sub):
                sl = pl.ds(u * 128, 128)
                cps.append(pltpu.async_copy(tc_hbm.at[ipv.at[u]], rci.at[sl], sem))
                cps.append(pltpu.async_copy(ta_hbm.at[ipv.at[u]], rxa.at[sl], sem))
                cps.append(pltpu.async_copy(tc_hbm.at[jpv.at[u]], rcj.at[sl], sem))
                cps.append(pltpu.async_copy(tb_hbm.at[jpv.at[u]], rxb.at[sl], sem))
            for cp in cps:
                cp.wait()
            pltpu.sync_copy(rci, ci_hbm.at[pl.ds(base, chunk)])
            pltpu.sync_copy(rxa, xa_hbm.at[pl.ds(base, chunk)])
            pltpu.sync_copy(rcj, cj_hbm.at[pl.ds(base, chunk)])
            pltpu.sync_copy(rxb, xb_hbm.at[pl.ds(base, chunk)])
            return carry

        lax.fori_loop(0, n_chunks, body, 0)

    return k


def _scatter_call(n, e_pad, d, chunk, feature_split):
    n_pad = n + 32
    zr = n_pad // _NS
    orow = n // _NS
    ept = e_pad // _NS if feature_split else e_pad // _NW
    n_chunks = ept // chunk
    sub = chunk // 128
    mesh = plsc.VectorSubcoreMesh(core_axis_name="c", subcore_axis_name="s")

    @functools.partial(
        pl.kernel,
        out_type=jax.ShapeDtypeStruct((_NC, n, d), jnp.float32),
        mesh=mesh,
        compiler_params=pltpu.CompilerParams(use_tc_tiling_on_sc=False),
        scratch_types=[
            pltpu.VMEM((sub, 128), jnp.int32),
            pltpu.VMEM((chunk, d), jnp.float32),
            pltpu.VMEM_SHARED((n_pad, d), jnp.float32),
            pltpu.SemaphoreType.DMA,
        ],
    )
    def k(data_hbm, ip_hbm, z_hbm, out_hbm, iv, dv, acc, sem):
        c = lax.axis_index("c")
        s = lax.axis_index("s")
        pltpu.sync_copy(z_hbm.at[pl.ds(s * zr, zr)], acc.at[pl.ds(s * zr, zr)])
        plsc.subcore_barrier()
        if feature_split:
            tb128 = s * (ept // 128)
        else:
            tb128 = (s * _NC + c) * (ept // 128)

        def body(kk, carry):
            b128 = tb128 + kk * sub
            base = b128 * 128
            pltpu.sync_copy(ip_hbm.at[pl.ds(b128, sub)], iv)
            if feature_split:
                pltpu.sync_copy(
                    data_hbm.at[pl.ds(base, chunk), pl.ds(c * d, d)], dv)
            else:
                pltpu.sync_copy(data_hbm.at[pl.ds(base, chunk)], dv)
            adds = []
            for u in range(sub):
                adds.append(pltpu.async_copy(
                    dv.at[pl.ds(u * 128, 128)], acc.at[iv.at[u]], sem, add=True))
            for a in adds:
                a.wait()
            return carry

        lax.fori_loop(0, n_chunks, body, 0)
        plsc.subcore_barrier()
        pltpu.sync_copy(acc.at[pl.ds(s * orow, orow)],
                        out_hbm.at[c].at[pl.ds(s * orow, orow)])

    return k


def _silu(v):
    return v * jax.nn.sigmoid(v)


def _ea_unfold(ea_blk, ps, qs):
    acc = None
    for s in range(8):
        t = jnp.dot(jnp.dot(ps[s], ea_blk, preferred_element_type=jnp.float32),
                    qs[s], preferred_element_type=jnp.float32)
        acc = t if acc is None else acc + t
    return acc


def _edge0_body(g0i, g0j, ea, ps, qs, wsel, wia, wjb, wdd, b1t, we2bd, b2t,
                wc1bd, bc1t, wc2s, bc2, sel, onep, out_m, out_v):
    gi = g0i[...]
    gj = g0j[...]
    diff = gi - gj
    d2 = diff * diff
    eau = _ea_unfold(ea[...], ps[...], qs[...])
    hpre = (jnp.dot(gi, wia[...], preferred_element_type=jnp.float32)
            + jnp.dot(gj, wjb[...], preferred_element_type=jnp.float32)
            + jnp.dot(d2, wdd[...], preferred_element_type=jnp.float32)
            + jnp.dot(eau, wsel[...], preferred_element_type=jnp.float32)
            + b1t[...])
    m = _silu(hpre)
    m = _silu(jnp.dot(m, we2bd[...], preferred_element_type=jnp.float32)
              + b2t[...])
    q = _silu(jnp.dot(m, wc1bd[...], preferred_element_type=jnp.float32)
              + bc1t[...])
    cwp = (jnp.dot(q, wc2s[...], preferred_element_type=jnp.float32)
           + bc2[...])
    rcw = diff * jnp.dot(cwp, sel[...], preferred_element_type=jnp.float32)
    out_m[...] = m
    out_v[...] = rcw + onep[...]


def _edge1_body(ci, xai, cj, xbj, ea, ps, qs, wsel, wdd, b1t, we2bd, b2t,
                wc1bd, bc1t, wc2s, bc2, sel, out):
    diff = ci[...] - cj[...]
    d2 = diff * diff
    eau = _ea_unfold(ea[...], ps[...], qs[...])
    hpre = (xai[...] + xbj[...]
            + jnp.dot(eau, wsel[...], preferred_element_type=jnp.float32)
            + jnp.dot(d2, wdd[...], preferred_element_type=jnp.float32)
            + b1t[...])
    m = _silu(hpre)
    m = _silu(jnp.dot(m, we2bd[...], preferred_element_type=jnp.float32)
              + b2t[...])
    q = _silu(jnp.dot(m, wc1bd[...], preferred_element_type=jnp.float32)
              + bc1t[...])
    cwp = (jnp.dot(q, wc2s[...], preferred_element_type=jnp.float32)
           + bc2[...])
    out[...] = diff * jnp.dot(cwp, sel[...],
                              preferred_element_type=jnp.float32)


def _wspec(shape):
    return pl.BlockSpec(shape, lambda i: tuple(0 for _ in shape))


def _edge0_call(e_pad, be, weights):
    grid = (e_pad // be,)
    bp = be // 16
    wspecs = [_wspec(w.shape) for w in weights]
    return pl.pallas_call(
        _edge0_body,
        grid=grid,
        in_specs=[pl.BlockSpec((bp, 128), lambda i: (i, 0)),
                  pl.BlockSpec((bp, 128), lambda i: (i, 0)),
                  pl.BlockSpec((bp // 8, 128), lambda i: (i, 0))] + wspecs,
        out_specs=[pl.BlockSpec((bp, 512), lambda i: (i, 0)),
                   pl.BlockSpec((bp, 128), lambda i: (i, 0))],
        out_shape=[jax.ShapeDtypeStruct((e_pad // 16, 512), jnp.float32),
                   jax.ShapeDtypeStruct((e_pad // 16, 128), jnp.float32)],
    )


def _edge1_call(e_pad, be, weights):
    grid = (e_pad // be,)
    bp = be // 16
    wspecs = [_wspec(w.shape) for w in weights]
    return pl.pallas_call(
        _edge1_body,
        grid=grid,
        in_specs=[pl.BlockSpec((bp, 128), lambda i: (i, 0)),
                  pl.BlockSpec((bp, 512), lambda i: (i, 0)),
                  pl.BlockSpec((bp, 128), lambda i: (i, 0)),
                  pl.BlockSpec((bp, 512), lambda i: (i, 0)),
                  pl.BlockSpec((bp // 8, 128), lambda i: (i, 0))] + wspecs,
        out_specs=pl.BlockSpec((bp, 128), lambda i: (i, 0)),
        out_shape=jax.ShapeDtypeStruct((e_pad // 16, 128), jnp.float32),
    )


def _node_body(x, pos, accm, accv, wn1f, wn1m_lo, wn1m_hi, bn1, wn2, bn2,
               w1a1, w1b1, c8, xa1, xb1, cnt):
    v = accv[0] + accv[1]
    c = jnp.maximum(v[:, 2:3], 1.0)
    cnt[...] = c
    coors1 = pos[...] + v[:, 0:2] / c
    c8[...] = jnp.concatenate(
        [coors1, jnp.zeros((coors1.shape[0], 6), jnp.float32)], axis=1)
    h = _silu(jnp.dot(x[...], wn1f[...], preferred_element_type=jnp.float32)
              + jnp.dot(accm[0], wn1m_lo[...],
                        preferred_element_type=jnp.float32)
              + jnp.dot(accm[1], wn1m_hi[...],
                        preferred_element_type=jnp.float32)
              + bn1[...])
    feats1 = jnp.dot(h, wn2[...], preferred_element_type=jnp.float32) + bn2[...]
    xa1[...] = jnp.dot(feats1, w1a1[...], preferred_element_type=jnp.float32)
    xb1[...] = jnp.dot(feats1, w1b1[...], preferred_element_type=jnp.float32)


def _node_call(n, bn, weights):
    grid = (n // bn,)
    wspecs = [_wspec(w.shape) for w in weights]
    return pl.pallas_call(
        _node_body,
        grid=grid,
        in_specs=[pl.BlockSpec((bn, 2), lambda i: (i, 0)),
                  pl.BlockSpec((bn, 2), lambda i: (i, 0)),
                  pl.BlockSpec((2, bn, 16), lambda i: (0, i, 0)),
                  pl.BlockSpec((2, bn, 8), lambda i: (0, i, 0))] + wspecs,
        out_specs=[pl.BlockSpec((bn, 8), lambda i: (i, 0)),
                   pl.BlockSpec((bn, 32), lambda i: (i, 0)),
                   pl.BlockSpec((bn, 32), lambda i: (i, 0)),
                   pl.BlockSpec((bn, 1), lambda i: (i, 0))],
        out_shape=[jax.ShapeDtypeStruct((n, 8), jnp.float32),
                   jax.ShapeDtypeStruct((n, 32), jnp.float32),
                   jax.ShapeDtypeStruct((n, 32), jnp.float32),
                   jax.ShapeDtypeStruct((n, 1), jnp.float32)],
    )


def _final_body(c8, acc, cnt, out):
    v = acc[0] + acc[1]
    out[...] = c8[:, 0:2] + v[:, 0:2] / cnt[...]


def _final_call(n, bn):
    grid = (n // bn,)
    return pl.pallas_call(
        _final_body,
        grid=grid,
        in_specs=[pl.BlockSpec((bn, 8), lambda i: (i, 0)),
                  pl.BlockSpec((2, bn, 8), lambda i: (0, i, 0)),
                  pl.BlockSpec((bn, 1), lambda i: (i, 0))],
        out_specs=pl.BlockSpec((bn, 2), lambda i: (i, 0)),
        out_shape=jax.ShapeDtypeStruct((n, 2), jnp.float32),
    )


def kernel(x, edge_index, edge_attr, batch, positions, params):
    n = x.shape[0]
    e = edge_index.shape[1]
    assert n % 32 == 0 and n % 16 == 0
    quantum = 32 * 2048
    e_pad = ((e + quantum - 1) // quantum) * quantum
    n_pad = n + 32

    f32 = jnp.float32
    i = edge_index[0]
    j = edge_index[1]
    pad_e = e_pad - e
    ip = jnp.concatenate([i, jnp.full((pad_e,), n, jnp.int32)])
    jp = jnp.concatenate([j, jnp.full((pad_e,), n, jnp.int32)])
    ip2 = ip.reshape(e_pad // 128, 128)
    jp2 = jp.reshape(e_pad // 128, 128)
    ea16 = jnp.pad(edge_attr.astype(f32).reshape(-1),
                   (0, pad_e)).reshape(e_pad // 128, 128)

    p0 = params["l0"]
    p1 = params["l1"]

    w1a0 = p0["W_e1"][0:2]
    w1b0 = p0["W_e1"][2:4]
    wd0 = p0["W_e1"][4:5]
    wa0 = p0["W_e1"][5:6]
    b10 = p0["b_e1"].reshape(1, 32)
    we20, b20 = p0["W_e2"], p0["b_e2"].reshape(1, 32)
    wc10, bc10 = p0["W_c1"], p0["b_c1"].reshape(1, 32)
    wc20, bc20 = p0["W_c2"], p0["b_c2"].reshape(1, 1)
    wn1f = p0["W_n1"][0:2]
    wn1m_lo = p0["W_n1"][2:18]
    wn1m_hi = p0["W_n1"][18:34]
    bn1 = p0["b_n1"].reshape(1, 32)
    wn2, bn2 = p0["W_n2"], p0["b_n2"].reshape(1, 32)

    w1a1 = p1["W_e1"][0:32]
    w1b1 = p1["W_e1"][32:64]
    wd1 = p1["W_e1"][64:65]
    wa1 = p1["W_e1"][65:66]
    b11 = p1["b_e1"].reshape(1, 32)
    we21, b21 = p1["W_e2"], p1["b_e2"].reshape(1, 32)
    wc11, bc11 = p1["W_c1"], p1["b_c1"].reshape(1, 32)
    wc21, bc21 = p1["W_c2"], p1["b_c2"].reshape(1, 1)

    eye16 = jnp.eye(16, dtype=f32)

    def bd(blk):
        return jnp.kron(eye16, blk)

    def t16(row):
        return jnp.tile(row, (1, 16))

    z8x32 = jnp.zeros((8, 32), f32)
    wia0 = bd(z8x32.at[2:4].set(w1a0))
    wjb0 = bd(z8x32.at[2:4].set(w1b0))
    wdd0 = bd(z8x32.at[0].set(wd0[0]).at[1].set(wd0[0]))
    wdd1 = bd(z8x32.at[0].set(wd1[0]).at[1].set(wd1[0]))
    sel = bd(jnp.array([[1, 1, 0, 0, 0, 0, 0, 0]], f32))
    onep = t16(jnp.array([[0, 0, 1, 0, 0, 0, 0, 0]], f32))
    psn = np.zeros((8, 128, 16), np.float32)
    qsn = np.zeros((8, 128, 16), np.float32)
    for s_ in range(8):
        for q_ in range(16):
            psn[s_, 8 * q_ + s_, q_] = 1.0
        for g_ in range(16):
            qsn[s_, 16 * s_ + g_, g_] = 1.0
    ps = jnp.asarray(psn)
    qs = jnp.asarray(qsn)
    wsel0 = bd(wa0)
    wsel1 = bd(wa1)

    t0 = jnp.pad(jnp.concatenate([positions.astype(f32), x.astype(f32)],
                                 axis=1), ((0, 32), (0, 4)))
    g0i, g0j = _gather0_call(n_pad, e_pad, 2048)(t0, ip2, jp2)

    w_edge0 = (ps, qs, wsel0, wia0, wjb0, wdd0, t16(b10), bd(we20), t16(b20),
               bd(wc10), t16(bc10), bd(wc20), bc20, sel, onep)
    m2p, v0p = _edge0_call(e_pad, 2048, w_edge0)(
        g0i.reshape(e_pad // 16, 128), g0j.reshape(e_pad // 16, 128),
        ea16, *w_edge0)
    m2 = m2p.reshape(e_pad, 32)
    v0 = v0p.reshape(e_pad, 8)

    z16 = jnp.zeros((n_pad, 16), f32)
    z8 = jnp.zeros((n_pad, 8), f32)
    accm = _scatter_call(n, e_pad, 16, 1024, True)(m2, ip2, z16)
    accv = _scatter_call(n, e_pad, 8, 2048, False)(v0, ip2, z8)

    w_node = (wn1f, wn1m_lo, wn1m_hi, bn1, wn2, bn2, w1a1, w1b1)
    c8, xa1, xb1, cnt = _node_call(n, 2000, w_node)(
        x.astype(f32), positions.astype(f32), accm, accv, *w_node)

    tc1 = jnp.pad(c8, ((0, 32), (0, 0)))
    ta1 = jnp.pad(xa1, ((0, 32), (0, 0)))
    tb1 = jnp.pad(xb1, ((0, 32), (0, 0)))
    ci, xai, cj, xbj = _gather1_call(n_pad, e_pad, 1024)(
        tc1, ta1, tb1, ip2, jp2)

    w_edge1 = (ps, qs, wsel1, wdd1, t16(b11), bd(we21), t16(b21), bd(wc11),
               t16(bc11), bd(wc21), bc21, sel)
    v1p = _edge1_call(e_pad, 2048, w_edge1)(
        ci.reshape(e_pad // 16, 128), xai.reshape(e_pad // 16, 512),
        cj.reshape(e_pad // 16, 128), xbj.reshape(e_pad // 16, 512),
        ea16, *w_edge1)
    v1 = v1p.reshape(e_pad, 8)

    acc1 = _scatter_call(n, e_pad, 8, 2048, False)(v1, ip2, z8)

    return _final_call(n, 2000)(c8, acc1, cnt)
